# baseline (speedup 1.0000x reference)
import functools
import numpy as np
import jax
import jax.numpy as jnp
from jax import lax
from jax.experimental import pallas as pl
from jax.experimental.pallas import tpu as pltpu
from jax.experimental.pallas import tpu_sc as plsc

Z = 32
KC = 1024


def _argmax_body(x_ref, locs_ref, logits_ref, idx_ref):
    x = x_ref[0]
    locs = locs_ref[...]
    logits = logits_ref[...]
    scores = lax.dot_general(
        x, locs, (((1,), (1,)), ((), ())),
        preferred_element_type=jnp.float32)
    m2 = jnp.sum(locs * locs, axis=1)
    bias = (-0.5) * m2 + float(Z) * logits[0]
    scores = scores + bias[None, :]
    idx_ref[...] = jnp.argmax(scores, axis=-1).astype(jnp.int32)


def _compute_idx(x, locs, logits):
    b, hw, zd = x.shape
    return pl.pallas_call(
        _argmax_body,
        grid=(b,),
        in_specs=[
            pl.BlockSpec((1, hw, zd), lambda i: (i, 0, 0)),
            pl.BlockSpec((KC, zd), lambda i: (0, 0)),
            pl.BlockSpec((1, KC), lambda i: (0, 0)),
        ],
        out_specs=pl.BlockSpec((hw,), lambda i: (i,)),
        out_shape=jax.ShapeDtypeStruct((b * hw,), jnp.int32),
    )(x, locs, logits.reshape(1, KC))


def _make_sc_gather(b, hw, d):
    info = plsc.get_sparse_core_info()
    nc, ns = info.num_cores, info.num_subcores
    nw = nc * ns
    b_total = b * hw
    assert b_total % (8 * nw) == 0 and hw % (b_total // nw) == 0
    b_per_w = b_total // nw
    w_per_row = hw // b_per_w
    mesh = plsc.VectorSubcoreMesh(core_axis_name="c", subcore_axis_name="s")

    @functools.partial(
        pl.kernel,
        mesh=mesh,
        out_type=jax.ShapeDtypeStruct((b, hw, d), jnp.float32),
        scratch_types=[
            pltpu.VMEM((b_per_w,), jnp.int32),
            pltpu.VMEM((b_per_w, d), jnp.float32),
            pltpu.VMEM_SHARED((KC, d), jnp.float32),
            pltpu.SemaphoreType.DMA,
        ],
        compiler_params=pltpu.CompilerParams(use_tc_tiling_on_sc=False),
    )
    def gather_kernel(table_hbm, idx_hbm, out_hbm, idx_v, rows_v, table_sh, sem):
        cid = lax.axis_index("c")
        sid = lax.axis_index("s")
        wid = sid * nc + cid
        base = wid * b_per_w

        @pl.when(sid == 0)
        def _():
            pltpu.sync_copy(table_hbm, table_sh)

        pltpu.sync_copy(idx_hbm.at[pl.ds(base, b_per_w)], idx_v)
        plsc.subcore_barrier()
        pltpu.async_copy(table_sh.at[idx_v], rows_v, sem).wait()
        row = wid // w_per_row
        col = (wid % w_per_row) * b_per_w
        pltpu.sync_copy(rows_v, out_hbm.at[row, pl.ds(col, b_per_w)])

    return gather_kernel


def kernel(x, locs, logits):
    b, hw, zd = x.shape
    idx = _compute_idx(x, locs, logits)
    return _make_sc_gather(b, hw, zd)(locs, idx)


# speedup vs baseline: 1.2324x; 1.2324x over previous
import functools
import numpy as np
import jax
import jax.numpy as jnp
from jax import lax
from jax.experimental import pallas as pl
from jax.experimental.pallas import tpu as pltpu
from jax.experimental.pallas import tpu_sc as plsc

Z = 32
KC = 1024


def _argmax_body(x_ref, locs_ref, logits_ref, idx_ref):
    x = x_ref[0]
    locs = locs_ref[...]
    logits = logits_ref[...]
    s = lax.dot_general(
        locs, x, (((1,), (1,)), ((), ())),
        preferred_element_type=jnp.float32)
    m2 = jnp.sum(locs * locs, axis=1)
    bias = (-0.5) * m2 + float(Z) * logits[0]
    s = s + bias[:, None]
    mx = jnp.max(s, axis=0)
    kio = lax.broadcasted_iota(jnp.int32, s.shape, 0).astype(jnp.float32)
    cand = jnp.where(s == mx[None, :], kio, float(KC))
    idx_ref[...] = jnp.min(cand, axis=0).astype(jnp.int32)


def _compute_idx(x, locs, logits):
    b, hw, zd = x.shape
    return pl.pallas_call(
        _argmax_body,
        grid=(b,),
        in_specs=[
            pl.BlockSpec((1, hw, zd), lambda i: (i, 0, 0)),
            pl.BlockSpec((KC, zd), lambda i: (0, 0)),
            pl.BlockSpec((1, KC), lambda i: (0, 0)),
        ],
        out_specs=pl.BlockSpec((hw,), lambda i: (i,)),
        out_shape=jax.ShapeDtypeStruct((b * hw,), jnp.int32),
    )(x, locs, logits.reshape(1, KC))


def _make_sc_gather(b, hw, d):
    info = plsc.get_sparse_core_info()
    nc, ns = info.num_cores, info.num_subcores
    nw = nc * ns
    b_total = b * hw
    assert b_total % (8 * nw) == 0 and hw % (b_total // nw) == 0
    b_per_w = b_total // nw
    w_per_row = hw // b_per_w
    mesh = plsc.VectorSubcoreMesh(core_axis_name="c", subcore_axis_name="s")

    @functools.partial(
        pl.kernel,
        mesh=mesh,
        out_type=jax.ShapeDtypeStruct((b, hw, d), jnp.float32),
        scratch_types=[
            pltpu.VMEM((b_per_w,), jnp.int32),
            pltpu.VMEM((b_per_w, d), jnp.float32),
            pltpu.VMEM_SHARED((KC, d), jnp.float32),
            pltpu.SemaphoreType.DMA,
        ],
        compiler_params=pltpu.CompilerParams(use_tc_tiling_on_sc=False),
    )
    def gather_kernel(table_hbm, idx_hbm, out_hbm, idx_v, rows_v, table_sh, sem):
        cid = lax.axis_index("c")
        sid = lax.axis_index("s")
        wid = sid * nc + cid
        base = wid * b_per_w

        @pl.when(sid == 0)
        def _():
            pltpu.sync_copy(table_hbm, table_sh)

        pltpu.sync_copy(idx_hbm.at[pl.ds(base, b_per_w)], idx_v)
        plsc.subcore_barrier()
        pltpu.async_copy(table_sh.at[idx_v], rows_v, sem).wait()
        row = wid // w_per_row
        col = (wid % w_per_row) * b_per_w
        pltpu.sync_copy(rows_v, out_hbm.at[row, pl.ds(col, b_per_w)])

    return gather_kernel


def kernel(x, locs, logits):
    b, hw, zd = x.shape
    idx = _compute_idx(x, locs, logits)
    return _make_sc_gather(b, hw, zd)(locs, idx)


# speedup vs baseline: 1.2804x; 1.0389x over previous
import functools
import numpy as np
import jax
import jax.numpy as jnp
from jax import lax
from jax.experimental import pallas as pl
from jax.experimental.pallas import tpu as pltpu
from jax.experimental.pallas import tpu_sc as plsc

Z = 32
KC = 1024


_ROWS_PER_BLOCK = 2


def _argmax_body(x_ref, locs_ref, logits_ref, idx_ref):
    xr = x_ref[...]
    x = xr.reshape(xr.shape[0] * xr.shape[1], xr.shape[2])
    locs = locs_ref[...]
    logits = logits_ref[...]
    s = lax.dot_general(
        locs, x, (((1,), (1,)), ((), ())),
        preferred_element_type=jnp.float32)
    m2 = jnp.sum(locs * locs, axis=1)
    bias = (-0.5) * m2 + float(Z) * logits[0]
    s = s + bias[:, None]
    mx = jnp.max(s, axis=0)
    kio = lax.broadcasted_iota(jnp.int32, s.shape, 0).astype(jnp.float32)
    cand = jnp.where(s == mx[None, :], kio, float(KC))
    idx_ref[...] = jnp.min(cand, axis=0).astype(jnp.int32)


def _compute_idx(x, locs, logits):
    b, hw, zd = x.shape
    r = _ROWS_PER_BLOCK
    return pl.pallas_call(
        _argmax_body,
        grid=(b // r,),
        in_specs=[
            pl.BlockSpec((r, hw, zd), lambda i: (i, 0, 0)),
            pl.BlockSpec((KC, zd), lambda i: (0, 0)),
            pl.BlockSpec((1, KC), lambda i: (0, 0)),
        ],
        out_specs=pl.BlockSpec((r * hw,), lambda i: (i,)),
        out_shape=jax.ShapeDtypeStruct((b * hw,), jnp.int32),
    )(x, locs, logits.reshape(1, KC))


def _make_sc_gather(b, hw, d):
    info = plsc.get_sparse_core_info()
    nc, ns = info.num_cores, info.num_subcores
    nw = nc * ns
    b_total = b * hw
    assert b_total % (8 * nw) == 0 and hw % (b_total // nw) == 0
    b_per_w = b_total // nw
    w_per_row = hw // b_per_w
    mesh = plsc.VectorSubcoreMesh(core_axis_name="c", subcore_axis_name="s")

    @functools.partial(
        pl.kernel,
        mesh=mesh,
        out_type=jax.ShapeDtypeStruct((b, hw, d), jnp.float32),
        scratch_types=[
            pltpu.VMEM((b_per_w,), jnp.int32),
            pltpu.VMEM((b_per_w, d), jnp.float32),
            pltpu.VMEM_SHARED((KC, d), jnp.float32),
            pltpu.SemaphoreType.DMA,
        ],
        compiler_params=pltpu.CompilerParams(use_tc_tiling_on_sc=False),
    )
    def gather_kernel(table_hbm, idx_hbm, out_hbm, idx_v, rows_v, table_sh, sem):
        cid = lax.axis_index("c")
        sid = lax.axis_index("s")
        wid = sid * nc + cid
        base = wid * b_per_w

        @pl.when(sid == 0)
        def _():
            pltpu.sync_copy(table_hbm, table_sh)

        pltpu.sync_copy(idx_hbm.at[pl.ds(base, b_per_w)], idx_v)
        plsc.subcore_barrier()
        pltpu.async_copy(table_sh.at[idx_v], rows_v, sem).wait()
        row = wid // w_per_row
        col = (wid % w_per_row) * b_per_w
        pltpu.sync_copy(rows_v, out_hbm.at[row, pl.ds(col, b_per_w)])

    return gather_kernel


def kernel(x, locs, logits):
    b, hw, zd = x.shape
    idx = _compute_idx(x, locs, logits)
    return _make_sc_gather(b, hw, zd)(locs, idx)


# speedup vs baseline: 1.2930x; 1.0099x over previous
import functools
import numpy as np
import jax
import jax.numpy as jnp
from jax import lax
from jax.experimental import pallas as pl
from jax.experimental.pallas import tpu as pltpu
from jax.experimental.pallas import tpu_sc as plsc

Z = 32
KC = 1024


_ROWS_PER_BLOCK = 4


def _argmax_body(x_ref, locs_ref, logits_ref, idx_ref):
    xr = x_ref[...]
    x = xr.reshape(xr.shape[0] * xr.shape[1], xr.shape[2])
    locs = locs_ref[...]
    logits = logits_ref[...]
    s = lax.dot_general(
        locs, x, (((1,), (1,)), ((), ())),
        preferred_element_type=jnp.float32)
    m2 = jnp.sum(locs * locs, axis=1)
    bias = (-0.5) * m2 + float(Z) * logits[0]
    s = s + bias[:, None]
    mx = jnp.max(s, axis=0)
    kio = lax.broadcasted_iota(jnp.int32, s.shape, 0).astype(jnp.float32)
    cand = jnp.where(s == mx[None, :], kio, float(KC))
    idx_ref[...] = jnp.min(cand, axis=0).astype(jnp.int32)


def _compute_idx(x, locs, logits):
    b, hw, zd = x.shape
    r = _ROWS_PER_BLOCK
    return pl.pallas_call(
        _argmax_body,
        grid=(b // r,),
        in_specs=[
            pl.BlockSpec((r, hw, zd), lambda i: (i, 0, 0)),
            pl.BlockSpec((KC, zd), lambda i: (0, 0)),
            pl.BlockSpec((1, KC), lambda i: (0, 0)),
        ],
        out_specs=pl.BlockSpec((r * hw,), lambda i: (i,)),
        out_shape=jax.ShapeDtypeStruct((b * hw,), jnp.int32),
    )(x, locs, logits.reshape(1, KC))


def _make_sc_gather(b, hw, d):
    info = plsc.get_sparse_core_info()
    nc, ns = info.num_cores, info.num_subcores
    nw = nc * ns
    b_total = b * hw
    assert b_total % (8 * nw) == 0 and hw % (b_total // nw) == 0
    b_per_w = b_total // nw
    w_per_row = hw // b_per_w
    mesh = plsc.VectorSubcoreMesh(core_axis_name="c", subcore_axis_name="s")

    @functools.partial(
        pl.kernel,
        mesh=mesh,
        out_type=jax.ShapeDtypeStruct((b, hw, d), jnp.float32),
        scratch_types=[
            pltpu.VMEM((b_per_w,), jnp.int32),
            pltpu.VMEM((b_per_w, d), jnp.float32),
            pltpu.VMEM_SHARED((KC, d), jnp.float32),
            pltpu.SemaphoreType.DMA,
        ],
        compiler_params=pltpu.CompilerParams(use_tc_tiling_on_sc=False),
    )
    def gather_kernel(table_hbm, idx_hbm, out_hbm, idx_v, rows_v, table_sh, sem):
        cid = lax.axis_index("c")
        sid = lax.axis_index("s")
        wid = sid * nc + cid
        base = wid * b_per_w

        @pl.when(sid == 0)
        def _():
            pltpu.sync_copy(table_hbm, table_sh)

        pltpu.sync_copy(idx_hbm.at[pl.ds(base, b_per_w)], idx_v)
        plsc.subcore_barrier()
        pltpu.async_copy(table_sh.at[idx_v], rows_v, sem).wait()
        row = wid // w_per_row
        col = (wid % w_per_row) * b_per_w
        pltpu.sync_copy(rows_v, out_hbm.at[row, pl.ds(col, b_per_w)])

    return gather_kernel


def kernel(x, locs, logits):
    b, hw, zd = x.shape
    idx = _compute_idx(x, locs, logits)
    return _make_sc_gather(b, hw, zd)(locs, idx)


# speedup vs baseline: 1.2942x; 1.0009x over previous
import functools
import numpy as np
import jax
import jax.numpy as jnp
from jax import lax
from jax.experimental import pallas as pl
from jax.experimental.pallas import tpu as pltpu
from jax.experimental.pallas import tpu_sc as plsc

Z = 32
KC = 1024


_ROWS_PER_BLOCK = 4


def _argmax_body(x_ref, locs_ref, logits_ref, idx_ref):
    xr = x_ref[...]
    x = xr.reshape(xr.shape[0] * xr.shape[1], xr.shape[2])
    locs = locs_ref[...]
    logits = logits_ref[...].reshape(1, KC)
    s = lax.dot_general(
        locs, x, (((1,), (1,)), ((), ())),
        preferred_element_type=jnp.float32)
    m2 = jnp.sum(locs * locs, axis=1)
    bias = (-0.5) * m2 + float(Z) * logits[0]
    s = s + bias[:, None]
    mx = jnp.max(s, axis=0)
    kio = lax.broadcasted_iota(jnp.int32, s.shape, 0).astype(jnp.float32)
    cand = jnp.where(s == mx[None, :], kio, float(KC))
    idx_ref[...] = jnp.min(cand, axis=0).astype(jnp.int32)


def _compute_idx(x, locs, logits):
    b, hw, zd = x.shape
    r = _ROWS_PER_BLOCK
    return pl.pallas_call(
        _argmax_body,
        grid=(b // r,),
        in_specs=[
            pl.BlockSpec((r, hw, zd), lambda i: (i, 0, 0)),
            pl.BlockSpec((KC, zd), lambda i: (0, 0)),
            pl.BlockSpec((KC,), lambda i: (0,)),
        ],
        out_specs=pl.BlockSpec((r * hw,), lambda i: (i,)),
        out_shape=jax.ShapeDtypeStruct((b * hw,), jnp.int32),
    )(x, locs, logits)


def _make_sc_gather(b, hw, d):
    info = plsc.get_sparse_core_info()
    nc, ns = info.num_cores, info.num_subcores
    nw = nc * ns
    b_total = b * hw
    assert b_total % (8 * nw) == 0 and hw % (b_total // nw) == 0
    b_per_w = b_total // nw
    w_per_row = hw // b_per_w
    mesh = plsc.VectorSubcoreMesh(core_axis_name="c", subcore_axis_name="s")

    @functools.partial(
        pl.kernel,
        mesh=mesh,
        out_type=jax.ShapeDtypeStruct((b_total, d), jnp.float32),
        scratch_types=[
            pltpu.VMEM((b_per_w,), jnp.int32),
            pltpu.VMEM((b_per_w, d), jnp.float32),
            pltpu.VMEM_SHARED((KC, d), jnp.float32),
            pltpu.SemaphoreType.DMA,
        ],
        compiler_params=pltpu.CompilerParams(use_tc_tiling_on_sc=False),
    )
    def gather_kernel(table_hbm, idx_hbm, out_hbm, idx_v, rows_v, table_sh, sem):
        cid = lax.axis_index("c")
        sid = lax.axis_index("s")
        wid = sid * nc + cid
        base = wid * b_per_w

        @pl.when(sid == 0)
        def _():
            pltpu.sync_copy(table_hbm, table_sh)

        pltpu.sync_copy(idx_hbm.at[pl.ds(base, b_per_w)], idx_v)
        plsc.subcore_barrier()
        pltpu.async_copy(table_sh.at[idx_v], rows_v, sem).wait()
        pltpu.sync_copy(rows_v, out_hbm.at[pl.ds(base, b_per_w)])

    return gather_kernel


def kernel(x, locs, logits):
    b, hw, zd = x.shape
    idx = _compute_idx(x, locs, logits)
    out = _make_sc_gather(b, hw, zd)(locs, idx)
    return out.reshape(b, hw, zd)


# speedup vs baseline: 1.4999x; 1.1589x over previous
import functools
import numpy as np
import jax
import jax.numpy as jnp
from jax import lax
from jax.experimental import pallas as pl
from jax.experimental.pallas import tpu as pltpu
from jax.experimental.pallas import tpu_sc as plsc

Z = 32
KC = 1024

_ROWS_PER_BLOCK = 4


def _argmax_body(xt_ref, locst_ref, logits_ref, idx_ref):
    locst = locst_ref[...]
    logits = logits_ref[...]
    m2 = jnp.sum(locst * locst, axis=0)
    bias = (-0.5) * m2 + float(Z) * logits
    locst_aug = jnp.concatenate([locst, bias[None, :]], axis=0)
    pieces = []
    for r in range(xt_ref.shape[0]):
        xt = xt_ref[r]
        ones = jnp.ones((1, xt.shape[1]), jnp.float32)
        xt_aug = jnp.concatenate([xt, ones], axis=0)
        s = lax.dot_general(
            locst_aug, xt_aug, (((0,), (0,)), ((), ())),
            preferred_element_type=jnp.float32)
        mx = jnp.max(s, axis=0)
        kio = lax.broadcasted_iota(jnp.int32, s.shape, 0).astype(jnp.float32)
        cand = jnp.where(s == mx[None, :], kio, float(KC))
        pieces.append(jnp.min(cand, axis=0).astype(jnp.int32))
    idx_ref[...] = jnp.concatenate(pieces, axis=0)


def _compute_idx(xt, locst, logits):
    b, zd, hw = xt.shape
    r = _ROWS_PER_BLOCK
    return pl.pallas_call(
        _argmax_body,
        grid=(b // r,),
        in_specs=[
            pl.BlockSpec((r, zd, hw), lambda i: (i, 0, 0)),
            pl.BlockSpec((zd, KC), lambda i: (0, 0)),
            pl.BlockSpec((KC,), lambda i: (0,)),
        ],
        out_specs=pl.BlockSpec((r * hw,), lambda i: (i,)),
        out_shape=jax.ShapeDtypeStruct((b * hw,), jnp.int32),
    )(xt, locst, logits)


def _make_sc_gather(b_total, d):
    info = plsc.get_sparse_core_info()
    nc, ns = info.num_cores, info.num_subcores
    nw = nc * ns
    assert b_total % (8 * nw) == 0
    b_per_w = b_total // nw
    mesh = plsc.VectorSubcoreMesh(core_axis_name="c", subcore_axis_name="s")

    @functools.partial(
        pl.kernel,
        mesh=mesh,
        out_type=jax.ShapeDtypeStruct((b_total, d), jnp.float32),
        scratch_types=[
            pltpu.VMEM((b_per_w,), jnp.int32),
            pltpu.VMEM((b_per_w, d), jnp.float32),
            pltpu.VMEM_SHARED((KC, d), jnp.float32),
            pltpu.SemaphoreType.DMA,
        ],
        compiler_params=pltpu.CompilerParams(use_tc_tiling_on_sc=False),
    )
    def gather_kernel(table_hbm, idx_hbm, out_hbm, idx_v, rows_v, table_sh, sem):
        cid = lax.axis_index("c")
        sid = lax.axis_index("s")
        wid = sid * nc + cid
        base = wid * b_per_w

        @pl.when(sid == 0)
        def _():
            pltpu.sync_copy(table_hbm, table_sh)

        pltpu.sync_copy(idx_hbm.at[pl.ds(base, b_per_w)], idx_v)
        plsc.subcore_barrier()
        pltpu.async_copy(table_sh.at[idx_v], rows_v, sem).wait()
        pltpu.sync_copy(rows_v, out_hbm.at[pl.ds(base, b_per_w)])

    return gather_kernel


def kernel(x, locs, logits):
    b, hw, zd = x.shape
    xt = jnp.swapaxes(x, 1, 2)
    locst = locs.T
    idx = _compute_idx(xt, locst, logits)
    out = _make_sc_gather(b * hw, zd)(locs, idx)
    return out.reshape(b, hw, zd)
